# Initial kernel scaffold; baseline (speedup 1.0000x reference)
import jax
import jax.numpy as jnp
from jax.experimental import pallas as pl


def kernel(mp_idx, mp_val):
    raise NotImplementedError("write your pallas kernel here")


# speedup vs baseline: 2.1910x; 2.1910x over previous
import functools

import jax
import jax.numpy as jnp
from jax import lax
from jax.experimental import pallas as pl
from jax.experimental.pallas import tpu as pltpu
from jax.experimental.pallas import tpu_sc as plsc

NC = 2
NS = 16
L = 16
NW = NC * NS

B = 262144
D = 128
ROWS_PER_W = B // NW
CH = 256
NCHUNK = ROWS_PER_W // CH


def _sc_body(idx_hbm, val_hbm, out_hbm, idxb, valb, outb):
    wid = lax.axis_index("s") * NC + lax.axis_index("c")
    base = wid * ROWS_PER_W
    lanes = lax.iota(jnp.int32, L)

    def chunk_body(ci, carry):
        row0 = base + ci * CH
        pltpu.sync_copy(idx_hbm.at[pl.ds(row0 * 4, CH * 4)], idxb)
        pltpu.sync_copy(val_hbm.at[pl.ds(row0 * 8, CH * 8)], valb)

        def block_body(rb, c2):
            r = rb * L + lanes
            pmt = plsc.load_gather(idxb, [r * 4 + 2])
            bft = plsc.load_gather(idxb, [r * 4 + 3])
            prem = plsc.load_gather(valb, [r * 8])
            p1f = (((pmt + 127) & 127) + 1).astype(jnp.float32)
            q = (bft + 127) & 127
            rD = r * D
            for j in range(D):
                val = prem * jnp.minimum(p1f, jnp.float32(j + 1))
                val = jnp.where(q >= j, val, jnp.float32(0.0))
                plsc.store_scatter(outb, [rD + j], val)
            return c2

        lax.fori_loop(0, CH // L, block_body, 0)
        pltpu.sync_copy(outb, out_hbm.at[pl.ds(row0 * D, CH * D)])
        return carry

    lax.fori_loop(0, NCHUNK, chunk_body, 0)


@jax.jit
def kernel(mp_idx, mp_val):
    mp_idx = mp_idx.astype(jnp.int32).reshape(-1)
    mp_val = mp_val.astype(jnp.float32).reshape(-1)
    mesh = plsc.VectorSubcoreMesh(core_axis_name="c", subcore_axis_name="s")
    f = pl.kernel(
        _sc_body,
        out_type=jax.ShapeDtypeStruct((B * D,), jnp.float32),
        mesh=mesh,
        scratch_types=[
            pltpu.VMEM((CH * 4,), jnp.int32),
            pltpu.VMEM((CH * 8,), jnp.float32),
            pltpu.VMEM((CH * D,), jnp.float32),
        ],
        compiler_params=pltpu.CompilerParams(needs_layout_passes=False),
    )
    return f(mp_idx, mp_val).reshape(B, D)


# speedup vs baseline: 2.4288x; 1.1085x over previous
import functools

import jax
import jax.numpy as jnp
from jax import lax
from jax.experimental import pallas as pl
from jax.experimental.pallas import tpu as pltpu
from jax.experimental.pallas import tpu_sc as plsc

NC = 2
NS = 16
L = 16
NW = NC * NS

B = 262144
D = 128
ROWS_PER_W = B // NW
CH = 256
NCHUNK = ROWS_PER_W // CH


def _sc_body(idx_hbm, val_hbm, out_hbm,
             idxb0, idxb1, valb0, valb1, outb0, outb1, sin_i, sin_v, sout):
    idxb = (idxb0, idxb1)
    valb = (valb0, valb1)
    outb = (outb0, outb1)
    wid = lax.axis_index("s") * NC + lax.axis_index("c")
    base = wid * ROWS_PER_W
    lanes = lax.iota(jnp.int32, L)

    def start_in(ci, par):
        row0 = base + ci * CH
        pltpu.async_copy(idx_hbm.at[pl.ds(row0 * 4, CH * 4)], idxb[par], sin_i.at[par])
        pltpu.async_copy(val_hbm.at[pl.ds(row0 * 8, CH * 8)], valb[par], sin_v.at[par])

    def wait_in(par):
        pltpu.make_async_copy(idx_hbm.at[pl.ds(0, CH * 4)], idxb[par], sin_i.at[par]).wait()
        pltpu.make_async_copy(val_hbm.at[pl.ds(0, CH * 8)], valb[par], sin_v.at[par]).wait()

    def compute_chunk(par):
        def block_body(rb, c2):
            r = rb * L + lanes
            pmt = plsc.load_gather(idxb[par], [r * 4 + 2])
            bft = plsc.load_gather(idxb[par], [r * 4 + 3])
            prem = plsc.load_gather(valb[par], [r * 8])
            p1f = (((pmt + 127) & 127) + 1).astype(jnp.float32)
            q = (bft + 127) & 127
            rD = r * D
            for j in range(D):
                val = prem * jnp.minimum(p1f, jnp.float32(j + 1))
                val = jnp.where(q >= j, val, jnp.float32(0.0))
                plsc.store_scatter(outb[par], [rD + j], val)
            return c2

        lax.fori_loop(0, CH // L, block_body, 0)

    def start_out(ci, par):
        row0 = base + ci * CH
        pltpu.async_copy(outb[par], out_hbm.at[pl.ds(row0 * D, CH * D)], sout.at[par])

    def wait_out(par):
        pltpu.make_async_copy(outb[par], out_hbm.at[pl.ds(0, CH * D)], sout.at[par]).wait()

    start_in(0, 0)
    start_in(1, 1)

    def pair_body(cp, carry):
        ci0 = cp * 2
        for par in range(2):
            ci = ci0 + par
            wait_in(par)

            @pl.when(cp > 0)
            def _():
                wait_out(par)

            compute_chunk(par)
            start_out(ci, par)

            @pl.when(ci + 2 < NCHUNK)
            def _():
                start_in(ci + 2, par)

        return carry

    lax.fori_loop(0, NCHUNK // 2, pair_body, 0)
    wait_out(0)
    wait_out(1)


@jax.jit
def kernel(mp_idx, mp_val):
    mp_idx = mp_idx.astype(jnp.int32).reshape(-1)
    mp_val = mp_val.astype(jnp.float32).reshape(-1)
    mesh = plsc.VectorSubcoreMesh(core_axis_name="c", subcore_axis_name="s")
    f = pl.kernel(
        _sc_body,
        out_type=jax.ShapeDtypeStruct((B * D,), jnp.float32),
        mesh=mesh,
        scratch_types=[
            pltpu.VMEM((CH * 4,), jnp.int32),
            pltpu.VMEM((CH * 4,), jnp.int32),
            pltpu.VMEM((CH * 8,), jnp.float32),
            pltpu.VMEM((CH * 8,), jnp.float32),
            pltpu.VMEM((CH * D,), jnp.float32),
            pltpu.VMEM((CH * D,), jnp.float32),
            pltpu.SemaphoreType.DMA((2,)),
            pltpu.SemaphoreType.DMA((2,)),
            pltpu.SemaphoreType.DMA((2,)),
        ],
        compiler_params=pltpu.CompilerParams(needs_layout_passes=False),
    )
    return f(mp_idx, mp_val).reshape(B, D)


# speedup vs baseline: 5.3556x; 2.2050x over previous
import functools

import jax
import jax.numpy as jnp
from jax import lax
from jax.experimental import pallas as pl
from jax.experimental.pallas import tpu as pltpu
from jax.experimental.pallas import tpu_sc as plsc

NC = 2
NS = 16
L = 16
NW = NC * NS

B = 262144
D = 128
ROWS_PER_W = B // NW
CH = 256
NCHUNK = ROWS_PER_W // CH


def _sc_body(idx_hbm, val_hbm, out_hbm,
             idxb0, idxb1, valb0, valb1, outb0, outb1, sin_i, sin_v, sout):
    idxb = (idxb0, idxb1)
    valb = (valb0, valb1)
    outb = (outb0, outb1)
    wid = lax.axis_index("s") * NC + lax.axis_index("c")
    base = wid * ROWS_PER_W
    lanes = lax.iota(jnp.int32, L)

    def start_in(ci, par):
        row0 = base + ci * CH
        pltpu.async_copy(idx_hbm.at[pl.ds(row0 * 4, CH * 4)], idxb[par], sin_i.at[par])
        pltpu.async_copy(val_hbm.at[pl.ds(row0 * 8, CH * 8)], valb[par], sin_v.at[par])

    def wait_in(par):
        pltpu.make_async_copy(idx_hbm.at[pl.ds(0, CH * 4)], idxb[par], sin_i.at[par]).wait()
        pltpu.make_async_copy(val_hbm.at[pl.ds(0, CH * 8)], valb[par], sin_v.at[par]).wait()

    jc1 = [(lanes + (k * L + 1)).astype(jnp.float32) for k in range(D // L)]

    _gd = lax.GatherDimensionNumbers(
        offset_dims=(), collapsed_slice_dims=(0,), start_index_map=(0,))

    def bcast(v, bi):
        return lax.gather(v, bi[:, None], _gd, (1,),
                          mode=lax.GatherScatterMode.PROMISE_IN_BOUNDS)

    def compute_chunk(par):
        def block_body(rb, c2):
            r = rb * L + lanes
            pmt = plsc.load_gather(idxb[par], [r * 4 + 2])
            bft = plsc.load_gather(idxb[par], [r * 4 + 3])
            prem = plsc.load_gather(valb[par], [r * 8])
            p1f = (((pmt + 127) & 127) + 1).astype(jnp.float32)
            q1f = (((bft + 127) & 127) + 1).astype(jnp.float32)
            rowbase = rb * (L * D)
            for i in range(L):
                bi = jnp.full((L,), i, jnp.int32)
                p1b = bcast(p1f, bi)
                q1b = bcast(q1f, bi)
                prb = bcast(prem, bi)
                for k in range(D // L):
                    val = prb * jnp.minimum(p1b, jc1[k])
                    val = jnp.where(jc1[k] <= q1b, val, jnp.float32(0.0))
                    outb[par][pl.ds(rowbase + (i * D + k * L), L)] = val
            return c2

        lax.fori_loop(0, CH // L, block_body, 0)

    def start_out(ci, par):
        row0 = base + ci * CH
        pltpu.async_copy(outb[par], out_hbm.at[pl.ds(row0 * D, CH * D)], sout.at[par])

    def wait_out(par):
        pltpu.make_async_copy(outb[par], out_hbm.at[pl.ds(0, CH * D)], sout.at[par]).wait()

    start_in(0, 0)
    start_in(1, 1)

    def pair_body(cp, carry):
        ci0 = cp * 2
        for par in range(2):
            ci = ci0 + par
            wait_in(par)

            @pl.when(cp > 0)
            def _():
                wait_out(par)

            compute_chunk(par)
            start_out(ci, par)

            @pl.when(ci + 2 < NCHUNK)
            def _():
                start_in(ci + 2, par)

        return carry

    lax.fori_loop(0, NCHUNK // 2, pair_body, 0)
    wait_out(0)
    wait_out(1)


@jax.jit
def kernel(mp_idx, mp_val):
    mp_idx = mp_idx.astype(jnp.int32).reshape(-1)
    mp_val = mp_val.astype(jnp.float32).reshape(-1)
    mesh = plsc.VectorSubcoreMesh(core_axis_name="c", subcore_axis_name="s")
    f = pl.kernel(
        _sc_body,
        out_type=jax.ShapeDtypeStruct((B * D,), jnp.float32),
        mesh=mesh,
        scratch_types=[
            pltpu.VMEM((CH * 4,), jnp.int32),
            pltpu.VMEM((CH * 4,), jnp.int32),
            pltpu.VMEM((CH * 8,), jnp.float32),
            pltpu.VMEM((CH * 8,), jnp.float32),
            pltpu.VMEM((CH * D,), jnp.float32),
            pltpu.VMEM((CH * D,), jnp.float32),
            pltpu.SemaphoreType.DMA((2,)),
            pltpu.SemaphoreType.DMA((2,)),
            pltpu.SemaphoreType.DMA((2,)),
        ],
        compiler_params=pltpu.CompilerParams(needs_layout_passes=False),
    )
    return f(mp_idx, mp_val).reshape(B, D)


# speedup vs baseline: 20.6322x; 3.8524x over previous
import jax
import jax.numpy as jnp
from jax import lax
from jax.experimental import pallas as pl
from jax.experimental.pallas import tpu as pltpu
from jax.experimental.pallas import tpu_sc as plsc

NC = 2
NS = 16
L = 16
NW = NC * NS

B = 262144
D = 128
ROWS_PER_W = B // NW
CH = 256
NCHUNK = ROWS_PER_W // CH


def _sc_body(pmt_hbm, bft_hbm, prem_hbm, out_hbm,
             pmtb0, pmtb1, bftb0, bftb1, premb0, premb1, outb0, outb1,
             sin, sout):
    pmtb = (pmtb0, pmtb1)
    bftb = (bftb0, bftb1)
    premb = (premb0, premb1)
    outb = (outb0, outb1)
    wid = lax.axis_index("s") * NC + lax.axis_index("c")
    base = wid * ROWS_PER_W
    lanes = lax.iota(jnp.int32, L)

    def start_in(ci, par):
        row0 = base + ci * CH
        sl = pl.ds(row0, CH)
        pltpu.async_copy(pmt_hbm.at[sl], pmtb[par], sin.at[par])
        pltpu.async_copy(bft_hbm.at[sl], bftb[par], sin.at[par])
        pltpu.async_copy(prem_hbm.at[sl], premb[par], sin.at[par])

    def wait_in(par):
        sl = pl.ds(0, CH)
        pltpu.make_async_copy(pmt_hbm.at[sl], pmtb[par], sin.at[par]).wait()
        pltpu.make_async_copy(bft_hbm.at[sl], bftb[par], sin.at[par]).wait()
        pltpu.make_async_copy(prem_hbm.at[sl], premb[par], sin.at[par]).wait()

    jc1 = [(lanes + (k * L + 1)).astype(jnp.float32) for k in range(D // L)]

    _gd = lax.GatherDimensionNumbers(
        offset_dims=(), collapsed_slice_dims=(0,), start_index_map=(0,))

    def bcast(v, bi):
        return lax.gather(v, bi[:, None], _gd, (1,),
                          mode=lax.GatherScatterMode.PROMISE_IN_BOUNDS)

    def compute_chunk(par):
        def block_body(rb, c2):
            sl = pl.ds(rb * L, L)
            pmt = pmtb[par][sl]
            bft = bftb[par][sl]
            prem = premb[par][sl]
            p1f = (((pmt + 127) & 127) + 1).astype(jnp.float32)
            q1f = (((bft + 127) & 127) + 1).astype(jnp.float32)
            rowbase = rb * (L * D)
            for i in range(L):
                bi = jnp.full((L,), i, jnp.int32)
                p1b = bcast(p1f, bi)
                q1b = bcast(q1f, bi)
                prb = bcast(prem, bi)
                for k in range(D // L):
                    val = prb * jnp.minimum(p1b, jc1[k])
                    val = jnp.where(jc1[k] <= q1b, val, jnp.float32(0.0))
                    outb[par][pl.ds(rowbase + (i * D + k * L), L)] = val
            return c2

        lax.fori_loop(0, CH // L, block_body, 0)

    def start_out(ci, par):
        row0 = base + ci * CH
        pltpu.async_copy(outb[par], out_hbm.at[pl.ds(row0 * D, CH * D)], sout.at[par])

    def wait_out(par):
        pltpu.make_async_copy(outb[par], out_hbm.at[pl.ds(0, CH * D)], sout.at[par]).wait()

    start_in(0, 0)
    start_in(1, 1)

    def pair_body(cp, carry):
        ci0 = cp * 2
        for par in range(2):
            ci = ci0 + par
            wait_in(par)

            @pl.when(cp > 0)
            def _():
                wait_out(par)

            compute_chunk(par)
            start_out(ci, par)

            @pl.when(ci + 2 < NCHUNK)
            def _():
                start_in(ci + 2, par)

        return carry

    lax.fori_loop(0, NCHUNK // 2, pair_body, 0)
    wait_out(0)
    wait_out(1)


@jax.jit
def kernel(mp_idx, mp_val):
    mp_idx = mp_idx.astype(jnp.int32)
    pmt = mp_idx[:, 2]
    bft = mp_idx[:, 3]
    prem = mp_val[:, 0].astype(jnp.float32)
    mesh = plsc.VectorSubcoreMesh(core_axis_name="c", subcore_axis_name="s")
    f = pl.kernel(
        _sc_body,
        out_type=jax.ShapeDtypeStruct((B * D,), jnp.float32),
        mesh=mesh,
        scratch_types=[
            pltpu.VMEM((CH,), jnp.int32),
            pltpu.VMEM((CH,), jnp.int32),
            pltpu.VMEM((CH,), jnp.int32),
            pltpu.VMEM((CH,), jnp.int32),
            pltpu.VMEM((CH,), jnp.float32),
            pltpu.VMEM((CH,), jnp.float32),
            pltpu.VMEM((CH * D,), jnp.float32),
            pltpu.VMEM((CH * D,), jnp.float32),
            pltpu.SemaphoreType.DMA((2,)),
            pltpu.SemaphoreType.DMA((2,)),
        ],
        compiler_params=pltpu.CompilerParams(needs_layout_passes=False),
    )
    return f(pmt, bft, prem).reshape(B, D)
